# baseline (speedup 1.0000x reference)
import functools

import jax
import jax.numpy as jnp
from jax import lax
from jax.experimental import pallas as pl
from jax.experimental.pallas import tpu as pltpu
from jax.experimental.pallas import tpu_sc as plsc

N = 10000
E = 320000
D_IN = 128
D_HID = 128
D_OUT = 16

NC = 2
NS = 16
L = 16
NW = NC * NS

K = 128
NCH = 80
EC = NCH * K
EP = NW * EC
NPAD = 10240
RT = NPAD // NS
ZR = 128

DEG_W = 16


@functools.cache
def _mesh():
    return plsc.VectorSubcoreMesh(
        core_axis_name="c", subcore_axis_name="s",
        num_cores=NC, num_subcores=NS)


def _zero_vmem_rows(buf, rows, width):
    def zrow(r, carry):
        for j in range(width // L):
            buf[r, pl.ds(j * L, L)] = jnp.zeros((L,), jnp.float32)
        return carry
    lax.fori_loop(0, rows, zrow, 0)


@functools.cache
def _make_edge_pass(D, CK, NB=2, e0_mode=False, stage_tab=False):
    NCHK = EC // CK
    assert NCHK % NB == 0
    scratch = [
        pltpu.VMEM((EC,), jnp.int32),
        pltpu.VMEM((NCHK, CK), jnp.int32),
        pltpu.VMEM_SHARED((NPAD, D), jnp.float32),
    ]
    scratch += [pltpu.VMEM((CK, D), jnp.float32) for _ in range(NB)]
    scratch += [pltpu.SemaphoreType.DMA for _ in range(NB)]
    if stage_tab:
        scratch.append(pltpu.VMEM_SHARED((NPAD, D), jnp.float32))

    @functools.partial(
        pl.kernel,
        out_type=jax.ShapeDtypeStruct((NC, NPAD, D), jnp.float32),
        mesh=_mesh(),
        compiler_params=pltpu.CompilerParams(use_tc_tiling_on_sc=False),
        scratch_types=scratch,
    )
    def kern(tab_hbm, src_hbm, dst_hbm, out_hbm, sidx, didx, acc, *rest):
        bufs = rest[:NB]
        sems = rest[NB:2 * NB]
        tab = rest[2 * NB] if stage_tab else tab_hbm
        cid = lax.axis_index("c")
        sid = lax.axis_index("s")
        wid = cid * NS + sid
        _zero_vmem_rows(bufs[-1], CK, D)
        r0 = sid * RT
        def zcp(i, carry):
            pltpu.sync_copy(bufs[-1], acc.at[pl.ds(r0 + i * CK, CK)])
            return carry
        lax.fori_loop(0, RT // CK, zcp, 0)
        pltpu.sync_copy(dst_hbm.at[wid], didx)
        if not e0_mode:
            pltpu.sync_copy(src_hbm.at[wid], sidx)
        if stage_tab:
            pltpu.sync_copy(tab_hbm.at[pl.ds(r0, RT)], tab.at[pl.ds(r0, RT)])
        plsc.subcore_barrier()
        if e0_mode:
            e0 = jnp.where(lax.iota(jnp.int32, L) == 0, 1.0, 0.0)
            def fill(r, carry):
                for j in range(D // L):
                    bufs[0][r, pl.ds(j * L, L)] = (
                        e0 if j == 0 else jnp.zeros((L,), jnp.float32))
                return carry
            lax.fori_loop(0, CK, fill, 0)
            FD = 8
            def sgrp(g, carry):
                for b in range(FD):
                    pltpu.async_copy(bufs[0], acc.at[didx.at[g * FD + b]],
                                     sems[0], add=True)
                for b in range(FD):
                    pltpu.make_async_copy(bufs[0], acc.at[didx.at[0]],
                                          sems[0]).wait()
                return carry
            lax.fori_loop(0, NCHK // FD, sgrp, 0)
        else:
            for b in range(NB):
                pltpu.async_copy(tab.at[sidx.at[pl.ds(b * CK, CK)]],
                                 bufs[b], sems[b])
            def grp(jg, carry):
                for b in range(NB):
                    j = jg * NB + b
                    pltpu.make_async_copy(tab_hbm.at[pl.ds(0, CK)], bufs[b],
                                          sems[b]).wait()
                    pltpu.sync_copy(bufs[b], acc.at[didx.at[j]], add=True)
                    @pl.when(j + NB < NCHK)
                    def _():
                        pltpu.async_copy(
                            tab.at[sidx.at[pl.ds((j + NB) * CK, CK)]],
                            bufs[b], sems[b])
                return carry
            lax.fori_loop(0, NCHK // NB, grp, 0)
        plsc.subcore_barrier()
        pltpu.sync_copy(acc.at[pl.ds(r0, RT)], out_hbm.at[cid, pl.ds(r0, RT)])
    return kern


def _dinv_from_deg(deg_blk):
    deg = deg_blk[0, :, 0] + deg_blk[1, :, 0] + 1.0
    return lax.rsqrt(deg)[:, None]


BM1 = 1024


def _tc1_body(x_ref, w_ref, deg_ref, o_ref):
    dinv = _dinv_from_deg(deg_ref[...])
    h = jnp.dot(x_ref[...], w_ref[...], preferred_element_type=jnp.float32)
    i = pl.program_id(0)
    row = i * BM1 + lax.broadcasted_iota(jnp.int32, (BM1, 1), 0)
    o_ref[...] = jnp.where(row < N, dinv * h, 0.0)


def _tc1(x, W1, degacc):
    return pl.pallas_call(
        _tc1_body,
        grid=(NPAD // BM1,),
        in_specs=[
            pl.BlockSpec((BM1, D_IN), lambda i: (i, 0)),
            pl.BlockSpec((D_IN, D_HID), lambda i: (0, 0)),
            pl.BlockSpec((NC, BM1, DEG_W), lambda i: (0, i, 0)),
        ],
        out_specs=pl.BlockSpec((BM1, D_HID), lambda i: (i, 0)),
        out_shape=jax.ShapeDtypeStruct((NPAD, D_HID), jnp.float32),
    )(x, W1, degacc)


def _tc2_body(acc_ref, h1p_ref, deg_ref, b1_ref, w2_ref, o_ref):
    dinv = _dinv_from_deg(deg_ref[...])
    s = acc_ref[0] + acc_ref[1] + h1p_ref[...]
    out1 = jnp.maximum(dinv * s + b1_ref[...], 0.0)
    h2 = jnp.dot(out1, w2_ref[...], preferred_element_type=jnp.float32)
    o_ref[...] = dinv * h2


def _tc2(acc1, h1p, degacc, b1, W2):
    return pl.pallas_call(
        _tc2_body,
        grid=(NPAD // BM1,),
        in_specs=[
            pl.BlockSpec((NC, BM1, D_HID), lambda i: (0, i, 0)),
            pl.BlockSpec((BM1, D_HID), lambda i: (i, 0)),
            pl.BlockSpec((NC, BM1, DEG_W), lambda i: (0, i, 0)),
            pl.BlockSpec((1, D_HID), lambda i: (0, 0)),
            pl.BlockSpec((D_HID, D_OUT), lambda i: (0, 0)),
        ],
        out_specs=pl.BlockSpec((BM1, D_OUT), lambda i: (i, 0)),
        out_shape=jax.ShapeDtypeStruct((NPAD, D_OUT), jnp.float32),
    )(acc1, h1p, degacc, b1, W2)


BM3 = 2000


def _tc3_body(acc_ref, h2p_ref, deg_ref, b2_ref, o_ref):
    dinv = _dinv_from_deg(deg_ref[...])
    s = acc_ref[0] + acc_ref[1] + h2p_ref[...]
    o_ref[...] = dinv * s + b2_ref[...]


def _tc3(acc2, h2p, degacc, b2):
    return pl.pallas_call(
        _tc3_body,
        grid=(N // BM3,),
        in_specs=[
            pl.BlockSpec((NC, BM3, D_OUT), lambda i: (0, i, 0)),
            pl.BlockSpec((BM3, D_OUT), lambda i: (i, 0)),
            pl.BlockSpec((NC, BM3, DEG_W), lambda i: (0, i, 0)),
            pl.BlockSpec((1, D_OUT), lambda i: (0, 0)),
        ],
        out_specs=pl.BlockSpec((BM3, D_OUT), lambda i: (i, 0)),
        out_shape=jax.ShapeDtypeStruct((N, D_OUT), jnp.float32),
    )(acc2, h2p, degacc, b2)


def kernel(x, edge_index, W1, b1, W2, b2):
    src = edge_index[0].astype(jnp.int32)
    dst = edge_index[1].astype(jnp.int32)
    pad = EP - E
    padv = (jnp.arange(pad, dtype=jnp.int32) % (NPAD - N)) + N
    src2 = jnp.concatenate([src, padv]).reshape(NW, EC)
    dstp = jnp.concatenate([dst, padv])
    dst_c32 = dstp.reshape(NW, EC // 32, 32)
    dst_c64 = dstp.reshape(NW, EC // 64, 64)
    dst_c128 = dstp.reshape(NW, EC // 128, 128)
    dummy_tab = jnp.zeros((8, DEG_W), jnp.float32)

    degacc = _make_edge_pass(DEG_W, 128, 1, True)(dummy_tab, src2, dst_c128)
    h1p = _tc1(x, W1, degacc)
    acc1 = _make_edge_pass(D_HID, 32, 5)(h1p, src2, dst_c32)
    h2p = _tc2(acc1, h1p, degacc, b1.reshape(1, D_HID), W2)
    acc2 = _make_edge_pass(D_OUT, 64, 8, False, True)(h2p, src2, dst_c64)
    return _tc3(acc2, h2p, degacc, b2.reshape(1, D_OUT))


# speedup vs baseline: 1.0112x; 1.0112x over previous
import functools

import jax
import jax.numpy as jnp
from jax import lax
from jax.experimental import pallas as pl
from jax.experimental.pallas import tpu as pltpu
from jax.experimental.pallas import tpu_sc as plsc

N = 10000
E = 320000
D_IN = 128
D_HID = 128
D_OUT = 16

NC = 2
NS = 16
L = 16
NW = NC * NS

K = 128
NCH = 80
EC = NCH * K
EP = NW * EC
NPAD = 10240
RT = NPAD // NS
ZR = 128

DEG_W = 16


@functools.cache
def _mesh():
    return plsc.VectorSubcoreMesh(
        core_axis_name="c", subcore_axis_name="s",
        num_cores=NC, num_subcores=NS)


def _zero_vmem_rows(buf, rows, width):
    def zrow(r, carry):
        for j in range(width // L):
            buf[r, pl.ds(j * L, L)] = jnp.zeros((L,), jnp.float32)
        return carry
    lax.fori_loop(0, rows, zrow, 0)


@functools.cache
def _make_edge_pass(D, CK, NB=2, e0_mode=False, stage_tab=False):
    NCHK = EC // CK
    assert NCHK % NB == 0
    scratch = [
        pltpu.VMEM((EC,), jnp.int32),
        pltpu.VMEM((NCHK, CK), jnp.int32),
        pltpu.VMEM_SHARED((NPAD, D), jnp.float32),
    ]
    scratch += [pltpu.VMEM((CK, D), jnp.float32) for _ in range(NB)]
    scratch += [pltpu.SemaphoreType.DMA for _ in range(NB)]
    if stage_tab:
        scratch.append(pltpu.VMEM_SHARED((NPAD, D), jnp.float32))

    @functools.partial(
        pl.kernel,
        out_type=jax.ShapeDtypeStruct((NC, NPAD, D), jnp.float32),
        mesh=_mesh(),
        compiler_params=pltpu.CompilerParams(use_tc_tiling_on_sc=False),
        scratch_types=scratch,
    )
    def kern(tab_hbm, src_hbm, dst_hbm, out_hbm, sidx, didx, acc, *rest):
        bufs = rest[:NB]
        sems = rest[NB:2 * NB]
        tab = rest[2 * NB] if stage_tab else tab_hbm
        cid = lax.axis_index("c")
        sid = lax.axis_index("s")
        wid = cid * NS + sid
        _zero_vmem_rows(bufs[-1], CK, D)
        r0 = sid * RT
        def zcp(i, carry):
            pltpu.sync_copy(bufs[-1], acc.at[pl.ds(r0 + i * CK, CK)])
            return carry
        lax.fori_loop(0, RT // CK, zcp, 0)
        pltpu.sync_copy(dst_hbm.at[wid], didx)
        if not e0_mode:
            pltpu.sync_copy(src_hbm.at[wid], sidx)
        if stage_tab:
            pltpu.sync_copy(tab_hbm.at[pl.ds(r0, RT)], tab.at[pl.ds(r0, RT)])
        plsc.subcore_barrier()
        if e0_mode:
            e0 = jnp.where(lax.iota(jnp.int32, L) == 0, 1.0, 0.0)
            def fill(r, carry):
                for j in range(D // L):
                    bufs[0][r, pl.ds(j * L, L)] = (
                        e0 if j == 0 else jnp.zeros((L,), jnp.float32))
                return carry
            lax.fori_loop(0, CK, fill, 0)
            FD = 8
            def sgrp(g, carry):
                for b in range(FD):
                    pltpu.async_copy(bufs[0], acc.at[didx.at[g * FD + b]],
                                     sems[0], add=True)
                for b in range(FD):
                    pltpu.make_async_copy(bufs[0], acc.at[didx.at[0]],
                                          sems[0]).wait()
                return carry
            lax.fori_loop(0, NCHK // FD, sgrp, 0)
        else:
            for b in range(NB):
                pltpu.async_copy(tab.at[sidx.at[pl.ds(b * CK, CK)]],
                                 bufs[b], sems[b])
            def grp(jg, carry):
                for b in range(NB):
                    j = jg * NB + b
                    pltpu.make_async_copy(tab_hbm.at[pl.ds(0, CK)], bufs[b],
                                          sems[b]).wait()
                    pltpu.sync_copy(bufs[b], acc.at[didx.at[j]], add=True)
                    @pl.when(j + NB < NCHK)
                    def _():
                        pltpu.async_copy(
                            tab.at[sidx.at[pl.ds((j + NB) * CK, CK)]],
                            bufs[b], sems[b])
                return carry
            lax.fori_loop(0, NCHK // NB, grp, 0)
        plsc.subcore_barrier()
        pltpu.sync_copy(acc.at[pl.ds(r0, RT)], out_hbm.at[cid, pl.ds(r0, RT)])
    return kern


def _dinv_from_deg(deg_blk):
    deg = deg_blk[0, :, 0] + deg_blk[1, :, 0] + 1.0
    return lax.rsqrt(deg)[:, None]


BM1 = 1024


def _tc1_body(x_ref, w_ref, deg_ref, o_ref):
    dinv = _dinv_from_deg(deg_ref[...])
    h = jnp.dot(x_ref[...], w_ref[...], preferred_element_type=jnp.float32)
    i = pl.program_id(0)
    row = i * BM1 + lax.broadcasted_iota(jnp.int32, (BM1, 1), 0)
    o_ref[...] = jnp.where(row < N, dinv * h, 0.0)


def _tc1(x, W1, degacc):
    return pl.pallas_call(
        _tc1_body,
        grid=(NPAD // BM1,),
        in_specs=[
            pl.BlockSpec((BM1, D_IN), lambda i: (i, 0)),
            pl.BlockSpec((D_IN, D_HID), lambda i: (0, 0)),
            pl.BlockSpec((NC, BM1, DEG_W), lambda i: (0, i, 0)),
        ],
        out_specs=pl.BlockSpec((BM1, D_HID), lambda i: (i, 0)),
        out_shape=jax.ShapeDtypeStruct((NPAD, D_HID), jnp.float32),
    )(x, W1, degacc)


def _tc2_body(acc_ref, h1p_ref, deg_ref, b1_ref, w2_ref, o_ref):
    dinv = _dinv_from_deg(deg_ref[...])
    s = acc_ref[0] + acc_ref[1] + h1p_ref[...]
    out1 = jnp.maximum(dinv * s + b1_ref[...], 0.0)
    h2 = jnp.dot(out1, w2_ref[...], preferred_element_type=jnp.float32)
    o_ref[...] = dinv * h2


def _tc2(acc1, h1p, degacc, b1, W2):
    return pl.pallas_call(
        _tc2_body,
        grid=(NPAD // BM1,),
        in_specs=[
            pl.BlockSpec((NC, BM1, D_HID), lambda i: (0, i, 0)),
            pl.BlockSpec((BM1, D_HID), lambda i: (i, 0)),
            pl.BlockSpec((NC, BM1, DEG_W), lambda i: (0, i, 0)),
            pl.BlockSpec((1, D_HID), lambda i: (0, 0)),
            pl.BlockSpec((D_HID, D_OUT), lambda i: (0, 0)),
        ],
        out_specs=pl.BlockSpec((BM1, D_OUT), lambda i: (i, 0)),
        out_shape=jax.ShapeDtypeStruct((NPAD, D_OUT), jnp.float32),
    )(acc1, h1p, degacc, b1, W2)


BM3 = 2000


def _tc3_body(acc_ref, h2p_ref, deg_ref, b2_ref, o_ref):
    dinv = _dinv_from_deg(deg_ref[...])
    s = acc_ref[0] + acc_ref[1] + h2p_ref[...]
    o_ref[...] = dinv * s + b2_ref[...]


def _tc3(acc2, h2p, degacc, b2):
    return pl.pallas_call(
        _tc3_body,
        grid=(N // BM3,),
        in_specs=[
            pl.BlockSpec((NC, BM3, D_OUT), lambda i: (0, i, 0)),
            pl.BlockSpec((BM3, D_OUT), lambda i: (i, 0)),
            pl.BlockSpec((NC, BM3, DEG_W), lambda i: (0, i, 0)),
            pl.BlockSpec((1, D_OUT), lambda i: (0, 0)),
        ],
        out_specs=pl.BlockSpec((BM3, D_OUT), lambda i: (i, 0)),
        out_shape=jax.ShapeDtypeStruct((N, D_OUT), jnp.float32),
    )(acc2, h2p, degacc, b2)


def kernel(x, edge_index, W1, b1, W2, b2):
    src = edge_index[0].astype(jnp.int32)
    dst = edge_index[1].astype(jnp.int32)
    pad = EP - E
    padv = (jnp.arange(pad, dtype=jnp.int32) % (NPAD - N)) + N
    src2 = jnp.concatenate([src, padv]).reshape(NW, EC)
    dstp = jnp.concatenate([dst, padv])
    dst_c40 = dstp.reshape(NW, EC // 40, 40)
    dst_c128 = dstp.reshape(NW, EC // 128, 128)
    dummy_tab = jnp.zeros((8, DEG_W), jnp.float32)

    degacc = _make_edge_pass(DEG_W, 128, 1, True)(dummy_tab, src2, dst_c128)
    h1p = _tc1(x, W1, degacc)
    acc1 = _make_edge_pass(D_HID, 40, 4)(h1p, src2, dst_c40)
    h2p = _tc2(acc1, h1p, degacc, b1.reshape(1, D_HID), W2)
    acc2 = _make_edge_pass(D_OUT, 128, 4, False, True)(h2p, src2, dst_c128)
    return _tc3(acc2, h2p, degacc, b2.reshape(1, D_OUT))
